# Initial kernel scaffold; baseline (speedup 1.0000x reference)
import jax
import jax.numpy as jnp
from jax.experimental import pallas as pl


def kernel(x, vocab):
    raise NotImplementedError("write your pallas kernel here")


# speedup vs baseline: 1.1028x; 1.1028x over previous
import functools

import jax
import jax.numpy as jnp
from jax import lax
from jax.experimental import pallas as pl
from jax.experimental.pallas import tpu as pltpu
from jax.experimental.pallas import tpu_sc as plsc

NUM_EMB = 1_000_000
DIM = 32
B_TOTAL = 16384 * 50
NC, NS = 2, 16
NW = NC * NS
CHUNK = 128
ROWS_W = B_TOTAL // NW
CHUNKS_W = ROWS_W // CHUNK
K = 8
GROUP_ROWS = K * CHUNK
GROUPS = CHUNKS_W // K

_mesh = plsc.VectorSubcoreMesh(
    core_axis_name="c", subcore_axis_name="s", num_cores=NC, num_subcores=NS
)


@functools.partial(
    pl.kernel,
    out_type=jax.ShapeDtypeStruct((B_TOTAL, DIM), jnp.float32),
    mesh=_mesh,
    scratch_types=[
        pltpu.VMEM((CHUNKS_W, CHUNK), jnp.int32),
        pltpu.VMEM((GROUP_ROWS, DIM), jnp.float32),
        pltpu.SemaphoreType.DMA,
    ],
    compiler_params=pltpu.CompilerParams(use_tc_tiling_on_sc=False),
)
def _emb_gather(idx_hbm, tab_hbm, out_hbm, idx_v, rows_v, gsem):
    wid = lax.axis_index("s") * NC + lax.axis_index("c")
    row0 = wid * CHUNKS_W
    base = wid * ROWS_W
    pltpu.sync_copy(idx_hbm.at[pl.ds(row0, CHUNKS_W), :], idx_v)

    @pl.loop(0, GROUPS)
    def _(g):
        cps = [
            pltpu.async_copy(
                tab_hbm.at[idx_v.at[g * K + j]],
                rows_v.at[pl.ds(j * CHUNK, CHUNK), :],
                gsem,
            )
            for j in range(K)
        ]
        for cp in cps:
            cp.wait()
        pltpu.sync_copy(
            rows_v, out_hbm.at[pl.ds(base + g * GROUP_ROWS, GROUP_ROWS), :]
        )


def kernel(x, vocab):
    idx = x.reshape(B_TOTAL // CHUNK, CHUNK).astype(jnp.int32)
    out = _emb_gather(idx, vocab)
    return out.reshape(*x.shape, DIM)


# speedup vs baseline: 1.1141x; 1.0102x over previous
import functools

import jax
import jax.numpy as jnp
from jax import lax
from jax.experimental import pallas as pl
from jax.experimental.pallas import tpu as pltpu
from jax.experimental.pallas import tpu_sc as plsc

NUM_EMB = 1_000_000
DIM = 32
B_TOTAL = 16384 * 50
NC, NS = 2, 16
NW = NC * NS
CHUNK = 128
ROWS_W = B_TOTAL // NW
CHUNKS_W = ROWS_W // CHUNK
K = 10
GROUP_ROWS = K * CHUNK
GROUPS = CHUNKS_W // K

_mesh = plsc.VectorSubcoreMesh(
    core_axis_name="c", subcore_axis_name="s", num_cores=NC, num_subcores=NS
)


@functools.partial(
    pl.kernel,
    out_type=jax.ShapeDtypeStruct((B_TOTAL, DIM), jnp.float32),
    mesh=_mesh,
    scratch_types=[
        pltpu.VMEM((CHUNKS_W, CHUNK), jnp.int32),
        pltpu.VMEM((2, GROUP_ROWS, DIM), jnp.float32),
        pltpu.SemaphoreType.DMA,
        pltpu.SemaphoreType.DMA,
    ],
    compiler_params=pltpu.CompilerParams(use_tc_tiling_on_sc=False),
)
def _emb_gather(idx_hbm, tab_hbm, out_hbm, idx_v, rows_v, gsem, osem):
    wid = lax.axis_index("s") * NC + lax.axis_index("c")
    row0 = wid * CHUNKS_W
    base = wid * ROWS_W
    pltpu.sync_copy(idx_hbm.at[pl.ds(row0, CHUNKS_W), :], idx_v)

    def fire(g, b):
        return [
            pltpu.async_copy(
                tab_hbm.at[idx_v.at[g * K + j]],
                rows_v.at[b, pl.ds(j * CHUNK, CHUNK), :],
                gsem,
            )
            for j in range(K)
        ]

    def out_slice(g):
        return out_hbm.at[pl.ds(base + g * GROUP_ROWS, GROUP_ROWS), :]

    fire(0, 0)

    @pl.loop(0, GROUPS, step=2)
    def _(gg):
        for b in range(2):
            g = gg + b
            @pl.when(g > 0)
            def _():
                pltpu.make_async_copy(
                    rows_v.at[1 - b], out_slice(g - 1), osem
                ).wait()

            @pl.when(g + 1 < GROUPS)
            def _():
                fire(g + 1, 1 - b)

            for j in range(K):
                pltpu.make_async_copy(
                    tab_hbm.at[idx_v.at[g * K + j]],
                    rows_v.at[b, pl.ds(j * CHUNK, CHUNK), :],
                    gsem,
                ).wait()
            pltpu.async_copy(rows_v.at[b], out_slice(g), osem)

    pltpu.make_async_copy(
        rows_v.at[(GROUPS - 1) % 2], out_slice(GROUPS - 1), osem
    ).wait()


def kernel(x, vocab):
    idx = x.reshape(B_TOTAL // CHUNK, CHUNK).astype(jnp.int32)
    out = _emb_gather(idx, vocab)
    return out.reshape(*x.shape, DIM)


# speedup vs baseline: 1.5001x; 1.3464x over previous
import functools

import jax
import jax.numpy as jnp
from jax import lax
from jax.experimental import pallas as pl
from jax.experimental.pallas import tpu as pltpu
from jax.experimental.pallas import tpu_sc as plsc

NUM_EMB = 1_000_000
DIM = 32
NI = 16384
NJ = 50
NC, NS = 2, 16
NW = NC * NS
IPW = NI // NW
CHUNK = 128
NCH = IPW // CHUNK
NK = IPW // 16

_mesh = plsc.VectorSubcoreMesh(
    core_axis_name="c", subcore_axis_name="s", num_cores=NC, num_subcores=NS
)


@functools.partial(
    pl.kernel,
    out_type=jax.ShapeDtypeStruct((NJ, DIM, NI), jnp.float32),
    mesh=_mesh,
    scratch_types=[
        pltpu.VMEM((NJ, IPW), jnp.int32),
        pltpu.VMEM((IPW, DIM), jnp.float32),
        pltpu.VMEM((IPW, DIM), jnp.float32),
        pltpu.VMEM((DIM, IPW), jnp.float32),
        pltpu.VMEM((DIM, IPW), jnp.float32),
        pltpu.SemaphoreType.DMA,
        pltpu.SemaphoreType.DMA,
    ],
    compiler_params=pltpu.CompilerParams(
        use_tc_tiling_on_sc=False, needs_layout_passes=False
    ),
)
def _emb_gather(xt_hbm, tab_hbm, out_hbm, idx_v, ra, rb, ta, tb, gsem, osem):
    wid = lax.axis_index("s") * NC + lax.axis_index("c")
    ib = wid * IPW
    rows = (ra, rb)
    trs = (ta, tb)
    pltpu.sync_copy(xt_hbm.at[:, pl.ds(ib, IPW)], idx_v)

    def fire(j, b):
        return [
            pltpu.async_copy(
                tab_hbm.at[idx_v.at[j, pl.ds(c * CHUNK, CHUNK)]],
                rows[b].at[pl.ds(c * CHUNK, CHUNK), :],
                gsem,
            )
            for c in range(NCH)
        ]

    def drain(j, b):
        for c in range(NCH):
            pltpu.make_async_copy(
                tab_hbm.at[idx_v.at[j, pl.ds(c * CHUNK, CHUNK)]],
                rows[b].at[pl.ds(c * CHUNK, CHUNK), :],
                gsem,
            ).wait()

    def out_slice(j):
        return out_hbm.at[j, :, pl.ds(ib, IPW)]

    k_iota = lax.iota(jnp.int32, 16)

    fire(0, 0)

    @pl.loop(0, NJ, step=2)
    def _(jj):
        for b in range(2):
            j = jj + b

            @pl.when(j + 1 < NJ)
            def _():
                fire(j + 1, 1 - b)

            drain(j, b)

            @pl.when(j >= 2)
            def _():
                pltpu.make_async_copy(trs[b], out_slice(j - 2), osem).wait()

            @pl.loop(0, DIM)
            def _(d):
                d_vec = jnp.full((16,), d, jnp.int32)
                for k0 in range(NK):
                    v = plsc.load_gather(
                        rows[b], [k_iota + (k0 * 16), d_vec]
                    )
                    trs[b][d, pl.ds(k0 * 16, 16)] = v

            pltpu.async_copy(trs[b], out_slice(j), osem)

    pltpu.make_async_copy(trs[0], out_slice(NJ - 2), osem).wait()
    pltpu.make_async_copy(trs[1], out_slice(NJ - 1), osem).wait()


def kernel(x, vocab):
    xt = x.T.astype(jnp.int32)
    out_t = _emb_gather(xt, vocab)
    return out_t.transpose(2, 0, 1)


# speedup vs baseline: 1.6434x; 1.0955x over previous
import functools

import jax
import jax.numpy as jnp
from jax import lax
from jax.experimental import pallas as pl
from jax.experimental.pallas import tpu as pltpu
from jax.experimental.pallas import tpu_sc as plsc

NUM_EMB = 1_000_000
DIM = 32
NI = 16384
NJ = 50
NC, NS = 2, 16
NW = NC * NS
IPW = NI // NW
CHUNK = 128
NCH = IPW // CHUNK
NK = IPW // 16

_mesh = plsc.VectorSubcoreMesh(
    core_axis_name="c", subcore_axis_name="s", num_cores=NC, num_subcores=NS
)


@functools.partial(
    pl.kernel,
    out_type=jax.ShapeDtypeStruct((NJ, DIM, NI), jnp.float32),
    mesh=_mesh,
    scratch_types=[
        pltpu.VMEM((NJ, IPW), jnp.int32),
        pltpu.VMEM((IPW, DIM), jnp.float32),
        pltpu.VMEM((IPW, DIM), jnp.float32),
        pltpu.VMEM((DIM, IPW), jnp.float32),
        pltpu.VMEM((DIM, IPW), jnp.float32),
        pltpu.SemaphoreType.DMA,
        pltpu.SemaphoreType.DMA,
    ],
    compiler_params=pltpu.CompilerParams(
        use_tc_tiling_on_sc=False, needs_layout_passes=False
    ),
)
def _emb_gather(xt_hbm, tab_hbm, out_hbm, idx_v, ra, rb, ta, tb, gsem, osem):
    wid = lax.axis_index("s") * NC + lax.axis_index("c")
    ib = wid * IPW
    rows = (ra, rb)
    trs = (ta, tb)
    pltpu.sync_copy(xt_hbm.at[:, pl.ds(ib, IPW)], idx_v)

    def fire(j, b):
        return [
            pltpu.async_copy(
                tab_hbm.at[idx_v.at[j, pl.ds(c * CHUNK, CHUNK)]],
                rows[b].at[pl.ds(c * CHUNK, CHUNK), :],
                gsem,
            )
            for c in range(NCH)
        ]

    def drain(j, b):
        for c in range(NCH):
            pltpu.make_async_copy(
                tab_hbm.at[idx_v.at[j, pl.ds(c * CHUNK, CHUNK)]],
                rows[b].at[pl.ds(c * CHUNK, CHUNK), :],
                gsem,
            ).wait()

    def out_slice(j):
        return out_hbm.at[j, :, pl.ds(ib, IPW)]

    k_iota = lax.iota(jnp.int32, 16)

    fire(0, 0)

    @pl.loop(0, NJ, step=2)
    def _(jj):
        for b in range(2):
            j = jj + b

            @pl.when(j + 1 < NJ)
            def _():
                fire(j + 1, 1 - b)

            drain(j, b)

            @pl.when(j >= 2)
            def _():
                pltpu.make_async_copy(trs[b], out_slice(j - 2), osem).wait()

            @pl.loop(0, DIM)
            def _(d):
                d_vec = jnp.full((16,), d, jnp.int32)
                for h in range(NK // 16):
                    vs = [
                        plsc.load_gather(
                            rows[b], [k_iota + ((h * 16 + k0) * 16), d_vec]
                        )
                        for k0 in range(16)
                    ]
                    for k0 in range(16):
                        trs[b][d, pl.ds((h * 16 + k0) * 16, 16)] = vs[k0]

            pltpu.async_copy(trs[b], out_slice(j), osem)

    pltpu.make_async_copy(trs[0], out_slice(NJ - 2), osem).wait()
    pltpu.make_async_copy(trs[1], out_slice(NJ - 1), osem).wait()


def kernel(x, vocab):
    xt = x.T.astype(jnp.int32)
    out_t = _emb_gather(xt, vocab)
    return out_t.transpose(2, 0, 1)


# speedup vs baseline: 1.6500x; 1.0041x over previous
import functools

import jax
import jax.numpy as jnp
from jax import lax
from jax.experimental import pallas as pl
from jax.experimental.pallas import tpu as pltpu
from jax.experimental.pallas import tpu_sc as plsc

NUM_EMB = 1_000_000
DIM = 32
NI = 16384
NJ = 50
NC, NS = 2, 16
NW = NC * NS
IPW = NI // NW
CHUNK = 128
NCH = IPW // CHUNK
NK = IPW // 16

_mesh = plsc.VectorSubcoreMesh(
    core_axis_name="c", subcore_axis_name="s", num_cores=NC, num_subcores=NS
)


@functools.partial(
    pl.kernel,
    out_type=jax.ShapeDtypeStruct((NJ, DIM, NI), jnp.float32),
    mesh=_mesh,
    scratch_types=[
        pltpu.VMEM((NJ, IPW), jnp.int32),
        pltpu.VMEM((2 * IPW, DIM), jnp.float32),
        pltpu.VMEM((2 * DIM, IPW), jnp.float32),
        pltpu.SemaphoreType.DMA,
        pltpu.SemaphoreType.DMA,
    ],
    compiler_params=pltpu.CompilerParams(
        use_tc_tiling_on_sc=False, needs_layout_passes=False
    ),
)
def _emb_gather(xt_hbm, tab_hbm, out_hbm, idx_v, rows_v, tr_v, gsem, osem):
    wid = lax.axis_index("s") * NC + lax.axis_index("c")
    ib = wid * IPW
    pltpu.sync_copy(xt_hbm.at[:, pl.ds(ib, IPW)], idx_v)

    def gather_cps(j, rbase):
        return [
            pltpu.make_async_copy(
                tab_hbm.at[idx_v.at[j, pl.ds(c * CHUNK, CHUNK)]],
                rows_v.at[pl.ds(rbase + c * CHUNK, CHUNK), :],
                gsem,
            )
            for c in range(NCH)
        ]

    def out_slice(j):
        return out_hbm.at[j, :, pl.ds(ib, IPW)]

    k_iota = lax.iota(jnp.int32, 16)

    for cp in gather_cps(0, 0):
        cp.start()

    @pl.loop(0, NJ)
    def _(j):
        b = j % 2
        rbase = b * IPW
        nrbase = IPW - rbase
        tbase = b * DIM

        @pl.when(j + 1 < NJ)
        def _():
            for cp in gather_cps(j + 1, nrbase):
                cp.start()

        for cp in gather_cps(j, rbase):
            cp.wait()

        @pl.when(j >= 2)
        def _():
            pltpu.make_async_copy(
                tr_v.at[pl.ds(tbase, DIM), :], out_slice(j - 2), osem
            ).wait()

        for d in range(DIM):
            d_vec = jnp.full((16,), d, jnp.int32)
            for h in range(NK // 16):
                vs = [
                    plsc.load_gather(
                        rows_v,
                        [rbase + k_iota + ((h * 16 + k0) * 16), d_vec],
                    )
                    for k0 in range(16)
                ]
                for k0 in range(16):
                    tr_v[tbase + d, pl.ds((h * 16 + k0) * 16, 16)] = vs[k0]

        pltpu.async_copy(tr_v.at[pl.ds(tbase, DIM), :], out_slice(j), osem)

    pltpu.make_async_copy(
        tr_v.at[pl.ds(0, DIM), :], out_slice(NJ - 2), osem
    ).wait()
    pltpu.make_async_copy(
        tr_v.at[pl.ds(DIM, DIM), :], out_slice(NJ - 1), osem
    ).wait()


def kernel(x, vocab):
    xt = x.T.astype(jnp.int32)
    out_t = _emb_gather(xt, vocab)
    return out_t.transpose(2, 0, 1)


# speedup vs baseline: 2.2043x; 1.3359x over previous
import functools

import jax
import jax.numpy as jnp
from jax import lax
from jax.experimental import pallas as pl
from jax.experimental.pallas import tpu as pltpu
from jax.experimental.pallas import tpu_sc as plsc

NUM_EMB = 1_000_000
DIM = 32
NI = 16384
NJ = 50
NC, NS = 2, 16
NW = NC * NS
IPW = NI // NW
CHUNK = 128
NCH = IPW // CHUNK
NK = IPW // 16
PITCH = IPW + 1

_mesh = plsc.VectorSubcoreMesh(
    core_axis_name="c", subcore_axis_name="s", num_cores=NC, num_subcores=NS
)


@functools.partial(
    pl.kernel,
    out_type=jax.ShapeDtypeStruct((NJ, DIM, NI), jnp.float32),
    mesh=_mesh,
    scratch_types=[
        pltpu.VMEM((NJ, IPW), jnp.int32),
        pltpu.VMEM((2 * IPW, DIM), jnp.float32),
        pltpu.VMEM((2 * DIM, PITCH), jnp.float32),
        pltpu.SemaphoreType.DMA,
        pltpu.SemaphoreType.DMA,
    ],
    compiler_params=pltpu.CompilerParams(
        use_tc_tiling_on_sc=False, needs_layout_passes=False
    ),
)
def _emb_gather(xt_hbm, tab_hbm, out_hbm, idx_v, rows_v, tr_v, gsem, osem):
    wid = lax.axis_index("s") * NC + lax.axis_index("c")
    ib = wid * IPW
    pltpu.sync_copy(xt_hbm.at[:, pl.ds(ib, IPW)], idx_v)

    def gather_cps(j, rbase):
        return [
            pltpu.make_async_copy(
                tab_hbm.at[idx_v.at[j, pl.ds(c * CHUNK, CHUNK)]],
                rows_v.at[pl.ds(rbase + c * CHUNK, CHUNK), :],
                gsem,
            )
            for c in range(NCH)
        ]

    def out_slice(j):
        return out_hbm.at[j, :, pl.ds(ib, IPW)]

    k_iota = lax.iota(jnp.int32, 16)

    for cp in gather_cps(0, 0):
        cp.start()

    @pl.loop(0, NJ)
    def _(j):
        b = j % 2
        rbase = b * IPW
        nrbase = IPW - rbase
        tbase = b * DIM

        @pl.when(j + 1 < NJ)
        def _():
            for cp in gather_cps(j + 1, nrbase):
                cp.start()

        for cp in gather_cps(j, rbase):
            cp.wait()

        @pl.when(j >= 2)
        def _():
            pltpu.make_async_copy(
                tr_v.at[pl.ds(tbase, DIM), pl.ds(0, IPW)], out_slice(j - 2), osem
            ).wait()

        for k in range(IPW):
            col = jnp.full((16,), k, jnp.int32)
            for half in range(2):
                val = rows_v[rbase + k, pl.ds(16 * half, 16)]
                row = k_iota + (tbase + 16 * half)
                plsc.store_scatter(tr_v, [row, col], val)

        pltpu.async_copy(
            tr_v.at[pl.ds(tbase, DIM), pl.ds(0, IPW)], out_slice(j), osem
        )

    pltpu.make_async_copy(
        tr_v.at[pl.ds(0, DIM), pl.ds(0, IPW)], out_slice(NJ - 2), osem
    ).wait()
    pltpu.make_async_copy(
        tr_v.at[pl.ds(DIM, DIM), pl.ds(0, IPW)], out_slice(NJ - 1), osem
    ).wait()


def kernel(x, vocab):
    xt = x.T.astype(jnp.int32)
    out_t = _emb_gather(xt, vocab)
    return out_t.transpose(2, 0, 1)
